# baseline (speedup 1.0000x reference)
import functools

import jax
import jax.numpy as jnp
from jax import lax
from jax.experimental import pallas as pl
from jax.experimental.pallas import tpu as pltpu
from jax.experimental.pallas import tpu_sc as plsc

B = 4096
C = 100
CT = 128
NG = CT // 16
NC = 2
NS = 16
L = 16
NW = NC * NS
RW = B // NW
KB = 8
NB = RW // KB

_LN2 = 0.6931471805599453
_C0 = -2.7868130207061768
_C1 = 5.046875953674316
_C2 = -3.4924943447113037
_C3 = 1.5939013957977295
_C4 = -0.40486717224121094
_C5 = 0.04342890903353691


def _log_ge1(u):
    bits = lax.bitcast_convert_type(u, jnp.int32)
    e = jnp.right_shift(bits, 23) - 127
    m = lax.bitcast_convert_type(
        jnp.bitwise_or(jnp.bitwise_and(bits, 0x007FFFFF), 0x3F800000),
        jnp.float32)
    p = _C5
    p = p * m + _C4
    p = p * m + _C3
    p = p * m + _C2
    p = p * m + _C1
    p = p * m + _C0
    return (e.astype(jnp.float32) + p) * _LN2


def _sc_body(q_hbm, out_hbm, q_v, o_v):
    wid = lax.axis_index("s") * NC + lax.axis_index("c")
    pltpu.sync_copy(q_hbm.at[pl.ds(wid * RW, RW), :], q_v)
    ones = jnp.ones((L,), jnp.float32)
    zeros = jnp.zeros((L,), jnp.float32)
    izeros = jnp.zeros((L,), jnp.int32)
    for g in range(NG):
        goff = g * L

        @plsc.parallel_loop(0, NB, 1, unroll=2,
                            carry=(ones, ones, izeros, izeros, izeros))
        def blk_body(blk, carry, goff=goff):
            pa, pp, ea, ep, cnt = carry
            for rr in range(KB):
                row = blk * KB + rr
                bits = q_v[row, pl.ds(goff, L)]
                yi = jnp.bitwise_and(bits, 1)
                x = lax.bitcast_convert_type(
                    jnp.bitwise_xor(bits, jnp.left_shift(yi, 31)),
                    jnp.float32)
                u = 1.0 + jnp.exp(x)
                pa = pa * u
                pp = pp * jnp.where(yi > 0, u, 1.0)
                cnt = cnt + yi
            ab = lax.bitcast_convert_type(pa, jnp.int32)
            pb = lax.bitcast_convert_type(pp, jnp.int32)
            ea = ea + jnp.right_shift(ab, 23)
            ep = ep + jnp.right_shift(pb, 23)
            pa = lax.bitcast_convert_type(
                jnp.bitwise_or(jnp.bitwise_and(ab, 0x007FFFFF), 0x3F800000),
                jnp.float32)
            pp = lax.bitcast_convert_type(
                jnp.bitwise_or(jnp.bitwise_and(pb, 0x007FFFFF), 0x3F800000),
                jnp.float32)
            return (pa, pp, ea, ep, cnt)

        pa, pp, ea, ep, cnt = blk_body
        sp = (ep - NB * 127).astype(jnp.float32) * _LN2 + _log_ge1(pp)
        la = (ea - NB * 127).astype(jnp.float32) * _LN2 + _log_ge1(pa)
        o_v[pl.ds(goff, L)] = sp
        o_v[pl.ds(CT + goff, L)] = la - sp
        o_v[pl.ds(2 * CT + goff, L)] = cnt.astype(jnp.float32)
    pltpu.sync_copy(o_v, out_hbm.at[wid])


_sc_phase = functools.partial(
    pl.kernel,
    mesh=plsc.VectorSubcoreMesh(core_axis_name="c", subcore_axis_name="s"),
    out_type=jax.ShapeDtypeStruct((NW, 3 * CT), jnp.float32),
    scratch_types=[
        pltpu.VMEM((RW, CT), jnp.int32),
        pltpu.VMEM((3 * CT,), jnp.float32),
    ],
)(_sc_body)


def _combine_body(part_ref, out_ref):
    x = part_ref[...]
    sums = jnp.sum(x, axis=0)
    sum_pos = sums[0:C]
    sum_neg = sums[CT:CT + C]
    n_pos = sums[2 * CT:2 * CT + C]
    n_neg = jnp.float32(B) - n_pos
    valid = jnp.logical_and(n_pos > 0.0, n_neg > 0.0)
    denom = jnp.where(valid, n_pos * n_neg, 1.0)
    loss_c = (n_neg * sum_pos + n_pos * sum_neg) / denom
    total = jnp.sum(jnp.where(valid, loss_c, 0.0))
    count = jnp.sum(jnp.where(valid, 1.0, 0.0))
    out_ref[...] = jnp.full((1, 1), total / count, jnp.float32)


_combine = pl.pallas_call(
    _combine_body,
    out_shape=jax.ShapeDtypeStruct((1, 1), jnp.float32),
)


def kernel(pred_y, true_y, c_nums):
    del c_nums
    pb = lax.bitcast_convert_type(pred_y, jnp.int32)
    qi = jnp.bitwise_or(jnp.bitwise_and(pb, jnp.int32(-2)),
                        true_y.astype(jnp.int32))
    partials = _sc_phase(jnp.pad(qi, ((0, 0), (0, CT - C))))
    return _combine(partials)[0, 0]


# speedup vs baseline: 1.0043x; 1.0043x over previous
import functools

import jax
import jax.numpy as jnp
from jax import lax
from jax.experimental import pallas as pl
from jax.experimental.pallas import tpu as pltpu
from jax.experimental.pallas import tpu_sc as plsc

B = 4096
C = 100
CT = 128
NG = CT // 16
NC = 2
NS = 16
L = 16
NW = NC * NS
RW = B // NW
KB = 8
NB = RW // KB

_LN2 = 0.6931471805599453
_C0 = -2.7868130207061768
_C1 = 5.046875953674316
_C2 = -3.4924943447113037
_C3 = 1.5939013957977295
_C4 = -0.40486717224121094
_C5 = 0.04342890903353691


def _log_ge1(u):
    bits = lax.bitcast_convert_type(u, jnp.int32)
    e = jnp.right_shift(bits, 23) - 127
    m = lax.bitcast_convert_type(
        jnp.bitwise_or(jnp.bitwise_and(bits, 0x007FFFFF), 0x3F800000),
        jnp.float32)
    p = _C5
    p = p * m + _C4
    p = p * m + _C3
    p = p * m + _C2
    p = p * m + _C1
    p = p * m + _C0
    return (e.astype(jnp.float32) + p) * _LN2


def _sc_body(q_hbm, out_hbm, q_v, o_v):
    wid = lax.axis_index("s") * NC + lax.axis_index("c")
    pltpu.sync_copy(q_hbm.at[pl.ds(wid * RW, RW), :], q_v)
    ones = jnp.ones((L,), jnp.float32)
    zeros = jnp.zeros((L,), jnp.float32)
    izeros = jnp.zeros((L,), jnp.int32)
    for g in range(NG):
        goff = g * L

        def blk_body(blk, carry, goff=goff):
            pa, pp, ea, ep, cnt = carry
            for rr in range(KB):
                row = blk * KB + rr
                bits = q_v[row, pl.ds(goff, L)]
                yi = jnp.bitwise_and(bits, 1)
                x = lax.bitcast_convert_type(
                    jnp.bitwise_xor(bits, jnp.left_shift(yi, 31)),
                    jnp.float32)
                u = 1.0 + jnp.exp(x)
                pa = pa * u
                pp = pp * jnp.where(yi > 0, u, 1.0)
                cnt = cnt + yi
            ab = lax.bitcast_convert_type(pa, jnp.int32)
            pb = lax.bitcast_convert_type(pp, jnp.int32)
            ea = ea + jnp.right_shift(ab, 23)
            ep = ep + jnp.right_shift(pb, 23)
            pa = lax.bitcast_convert_type(
                jnp.bitwise_or(jnp.bitwise_and(ab, 0x007FFFFF), 0x3F800000),
                jnp.float32)
            pp = lax.bitcast_convert_type(
                jnp.bitwise_or(jnp.bitwise_and(pb, 0x007FFFFF), 0x3F800000),
                jnp.float32)
            return (pa, pp, ea, ep, cnt)

        pa, pp, ea, ep, cnt = lax.fori_loop(
            0, NB, blk_body, (ones, ones, izeros, izeros, izeros))
        sp = (ep - NB * 127).astype(jnp.float32) * _LN2 + _log_ge1(pp)
        la = (ea - NB * 127).astype(jnp.float32) * _LN2 + _log_ge1(pa)
        o_v[pl.ds(goff, L)] = sp
        o_v[pl.ds(CT + goff, L)] = la - sp
        o_v[pl.ds(2 * CT + goff, L)] = cnt.astype(jnp.float32)
    pltpu.sync_copy(o_v, out_hbm.at[wid])


_sc_phase = functools.partial(
    pl.kernel,
    mesh=plsc.VectorSubcoreMesh(core_axis_name="c", subcore_axis_name="s"),
    out_type=jax.ShapeDtypeStruct((NW, 3 * CT), jnp.float32),
    scratch_types=[
        pltpu.VMEM((RW, CT), jnp.int32),
        pltpu.VMEM((3 * CT,), jnp.float32),
    ],
)(_sc_body)


def _combine_body(part_ref, out_ref):
    x = part_ref[...]
    sums = jnp.sum(x, axis=0)
    sum_pos = sums[0:C]
    sum_neg = sums[CT:CT + C]
    n_pos = sums[2 * CT:2 * CT + C]
    n_neg = jnp.float32(B) - n_pos
    valid = jnp.logical_and(n_pos > 0.0, n_neg > 0.0)
    denom = jnp.where(valid, n_pos * n_neg, 1.0)
    loss_c = (n_neg * sum_pos + n_pos * sum_neg) / denom
    total = jnp.sum(jnp.where(valid, loss_c, 0.0))
    count = jnp.sum(jnp.where(valid, 1.0, 0.0))
    out_ref[...] = jnp.full((1, 1), total / count, jnp.float32)


_combine = pl.pallas_call(
    _combine_body,
    out_shape=jax.ShapeDtypeStruct((1, 1), jnp.float32),
)


def kernel(pred_y, true_y, c_nums):
    del c_nums
    pb = lax.bitcast_convert_type(pred_y, jnp.int32)
    qi = jnp.bitwise_or(jnp.bitwise_and(pb, jnp.int32(-2)),
                        true_y.astype(jnp.int32))
    partials = _sc_phase(jnp.pad(qi, ((0, 0), (0, CT - C))))
    return _combine(partials)[0, 0]
